# baseline (speedup 1.0000x reference)
import functools

import jax
import jax.numpy as jnp
from jax import lax
from jax.experimental import pallas as pl
from jax.experimental.pallas import tpu as pltpu
from jax.experimental.pallas import tpu_sc as plsc

VOCAB = 50265
MAXPOS = 8194
H = 1024
B = 4
S = 8192
N = B * S
EPS = 1e-5

NC = 2
NS = 16
NW = NC * NS
TPW = N // NW
G = 16
NCHUNK = TPW // G
NSLOT = 2
HV = H // 16

RB = 1024


def _sc_body(ids_hbm, pids_hbm, word_hbm, pos_hbm, out_hbm,
             idx_v, pidx_v, wbuf, pbuf, obuf,
             wsem0, wsem1, psem0, psem1, osem0, osem1,
             tpw, nchunk):
    wid = lax.axis_index("s") * NC + lax.axis_index("c")
    base = wid * tpw
    pltpu.sync_copy(ids_hbm.at[pl.ds(base, tpw)], idx_v)
    pltpu.sync_copy(pids_hbm.at[pl.ds(base, tpw)], pidx_v)

    wsem = [wsem0, wsem1]
    psem = [psem0, psem1]
    osem = [osem0, osem1]

    def wcp(c, b):
        off = pl.multiple_of(c * G, G)
        return pltpu.make_async_copy(
            word_hbm.at[idx_v.at[pl.ds(off, G)]], wbuf.at[b], wsem[b])

    def pcp(c, b):
        off = pl.multiple_of(c * G, G)
        return pltpu.make_async_copy(
            pos_hbm.at[pidx_v.at[pl.ds(off, G)]], pbuf.at[b], psem[b])

    def ocp(c, b):
        off = pl.multiple_of(c * G, G)
        return pltpu.make_async_copy(
            obuf.at[b], out_hbm.at[pl.ds(base + off, G)], osem[b])

    for b in range(NSLOT):
        wcp(b, b).start()
        pcp(b, b).start()

    def outer(g, _):
        for b in range(NSLOT):
            c = g * NSLOT + b
            wb = wbuf.at[b]
            pb = pbuf.at[b]
            ob = obuf.at[b]
            wcp(c, b).wait()
            pcp(c, b).wait()

            @pl.when(c >= NSLOT)
            def _w():
                ocp(c - NSLOT, b).wait()

            def tok_add(t, _):
                for h in range(HV):
                    hs = pl.ds(h * 16, 16)
                    ob[t, hs] = wb[t, hs] + pb[t, hs]
                return _

            lax.fori_loop(0, G, tok_add, None)
            ocp(c, b).start()

            @pl.when(c + NSLOT < nchunk)
            def _n():
                wcp(c + NSLOT, b).start()
                pcp(c + NSLOT, b).start()
        return _

    lax.fori_loop(0, nchunk // NSLOT, outer, None)
    for b in range(NSLOT):
        ocp(nchunk - NSLOT + b, b).wait()


def _ln_body(w_ref, b_ref, x_ref, o_ref):
    x = x_ref[...]
    m = jnp.mean(x, axis=-1, keepdims=True)
    d = x - m
    v = jnp.mean(d * d, axis=-1, keepdims=True)
    o_ref[...] = d * (lax.rsqrt(v + EPS) * w_ref[...]) + b_ref[...]


NSLICE = 2
NSL = N // NSLICE
TPW_S = NSL // NW
NCHUNK_S = TPW_S // G


def _make_gather(ntok):
    tpw = ntok // NW
    mesh = plsc.VectorSubcoreMesh(core_axis_name="c", subcore_axis_name="s")
    return pl.kernel(
        functools.partial(_sc_body, tpw=tpw, nchunk=tpw // G),
        out_type=jax.ShapeDtypeStruct((ntok, H), jnp.float32),
        mesh=mesh,
        scratch_types=[
            pltpu.VMEM((tpw,), jnp.int32),
            pltpu.VMEM((tpw,), jnp.int32),
            pltpu.VMEM((NSLOT, G, H), jnp.float32),
            pltpu.VMEM((NSLOT, G, H), jnp.float32),
            pltpu.VMEM((NSLOT, G, H), jnp.float32),
            pltpu.SemaphoreType.DMA,
            pltpu.SemaphoreType.DMA,
            pltpu.SemaphoreType.DMA,
            pltpu.SemaphoreType.DMA,
            pltpu.SemaphoreType.DMA,
            pltpu.SemaphoreType.DMA,
        ],
    )


def _ln_slice_body(w_ref, b_ref, x_ref, dst_ref, o_ref):
    del dst_ref
    _ln_body(w_ref, b_ref, x_ref, o_ref)


def _make_ln(sl, aliased):
    nblk = NSL // RB
    in_specs = [
        pl.BlockSpec((1, H), lambda i: (0, 0)),
        pl.BlockSpec((1, H), lambda i: (0, 0)),
        pl.BlockSpec((RB, H), lambda i: (i, 0)),
        pl.BlockSpec(memory_space=pl.ANY),
    ]
    return pl.pallas_call(
        _ln_slice_body,
        out_shape=jax.ShapeDtypeStruct((N, H), jnp.float32),
        grid=(nblk,),
        in_specs=in_specs,
        out_specs=pl.BlockSpec((RB, H), lambda i, s=sl: (s * nblk + i, 0)),
        input_output_aliases={3: 0} if aliased else {},
    )


@jax.jit
def _run(ids, pids, word, pos2, w2, b2):
    gather_k = _make_gather(NSL)
    w2r = w2.reshape(1, H)
    b2r = b2.reshape(1, H)
    dst = jnp.zeros((8, 128), jnp.float32)
    for sl in range(NSLICE):
        tok = slice(sl * NSL, (sl + 1) * NSL)
        summed = gather_k(ids[tok], pids[tok], word, pos2)
        dst = _make_ln(sl, sl > 0)(w2r, b2r, summed, dst)
    return dst


def kernel(input_ids, position_ids, word_emb, pos_emb, tok_emb, ln_w, ln_b, fed_mask):
    ids = input_ids.reshape(-1).astype(jnp.int32)
    pids = position_ids.reshape(-1).astype(jnp.int32)
    pos2 = pos_emb + tok_emb[0]
    w2 = ln_w * fed_mask
    b2 = ln_b * fed_mask
    out = _run(ids, pids, word_emb, pos2, w2, b2)
    return out.reshape(B, S, H)


# speedup vs baseline: 1.0253x; 1.0253x over previous
import functools

import jax
import jax.numpy as jnp
from jax import lax
from jax.experimental import pallas as pl
from jax.experimental.pallas import tpu as pltpu
from jax.experimental.pallas import tpu_sc as plsc

VOCAB = 50265
MAXPOS = 8194
H = 1024
B = 4
S = 8192
N = B * S
EPS = 1e-5

NC = 2
NS = 16
NW = NC * NS
TPW = N // NW
G = 16
NCHUNK = TPW // G
NSLOT = 2
HV = H // 16

RB = 2048


def _sc_body(ids_hbm, pids_hbm, word_hbm, pos_hbm, out_hbm,
             idx_v, pidx_v, wbuf, pbuf, obuf,
             wsem0, wsem1, psem0, psem1, osem0, osem1):
    wid = lax.axis_index("s") * NC + lax.axis_index("c")
    base = wid * TPW
    pltpu.sync_copy(ids_hbm.at[pl.ds(base, TPW)], idx_v)
    pltpu.sync_copy(pids_hbm.at[pl.ds(base, TPW)], pidx_v)

    wsem = [wsem0, wsem1]
    psem = [psem0, psem1]
    osem = [osem0, osem1]

    def wcp(c, b):
        off = pl.multiple_of(c * G, G)
        return pltpu.make_async_copy(
            word_hbm.at[idx_v.at[pl.ds(off, G)]], wbuf.at[b], wsem[b])

    def pcp(c, b):
        off = pl.multiple_of(c * G, G)
        return pltpu.make_async_copy(
            pos_hbm.at[pidx_v.at[pl.ds(off, G)]], pbuf.at[b], psem[b])

    def ocp(c, b):
        off = pl.multiple_of(c * G, G)
        return pltpu.make_async_copy(
            obuf.at[b], out_hbm.at[pl.ds(base + off, G)], osem[b])

    for b in range(NSLOT):
        wcp(b, b).start()
        pcp(b, b).start()

    def outer(g, _):
        for b in range(NSLOT):
            c = g * NSLOT + b
            wb = wbuf.at[b]
            pb = pbuf.at[b]
            ob = obuf.at[b]
            wcp(c, b).wait()
            pcp(c, b).wait()

            @pl.when(c >= NSLOT)
            def _w():
                ocp(c - NSLOT, b).wait()

            def tok_add(t, _):
                for h in range(HV):
                    hs = pl.ds(h * 16, 16)
                    ob[t, hs] = wb[t, hs] + pb[t, hs]
                return _

            lax.fori_loop(0, G, tok_add, None)
            ocp(c, b).start()

            @pl.when(c + NSLOT < NCHUNK)
            def _n():
                wcp(c + NSLOT, b).start()
                pcp(c + NSLOT, b).start()
        return _

    lax.fori_loop(0, NCHUNK // NSLOT, outer, None)
    for b in range(NSLOT):
        ocp(NCHUNK - NSLOT + b, b).wait()


def _ln_body(w_ref, b_ref, x_ref, o_ref):
    x = x_ref[...]
    m = jnp.mean(x, axis=-1, keepdims=True)
    d = x - m
    v = jnp.mean(d * d, axis=-1, keepdims=True)
    o_ref[...] = d * (lax.rsqrt(v + EPS) * w_ref[...]) + b_ref[...]


@jax.jit
def _run(ids, pids, word, pos2, w2, b2):
    mesh = plsc.VectorSubcoreMesh(core_axis_name="c", subcore_axis_name="s")
    gather_k = pl.kernel(
        _sc_body,
        out_type=jax.ShapeDtypeStruct((N, H), jnp.float32),
        mesh=mesh,
        scratch_types=[
            pltpu.VMEM((TPW,), jnp.int32),
            pltpu.VMEM((TPW,), jnp.int32),
            pltpu.VMEM((NSLOT, G, H), jnp.float32),
            pltpu.VMEM((NSLOT, G, H), jnp.float32),
            pltpu.VMEM((NSLOT, G, H), jnp.float32),
            pltpu.SemaphoreType.DMA,
            pltpu.SemaphoreType.DMA,
            pltpu.SemaphoreType.DMA,
            pltpu.SemaphoreType.DMA,
            pltpu.SemaphoreType.DMA,
            pltpu.SemaphoreType.DMA,
        ],
    )
    summed = gather_k(ids, pids, word, pos2)

    ln = pl.pallas_call(
        _ln_body,
        out_shape=jax.ShapeDtypeStruct((N, H), jnp.float32),
        grid=(N // RB,),
        in_specs=[
            pl.BlockSpec((1, H), lambda i: (0, 0)),
            pl.BlockSpec((1, H), lambda i: (0, 0)),
            pl.BlockSpec((RB, H), lambda i: (i, 0)),
        ],
        out_specs=pl.BlockSpec((RB, H), lambda i: (i, 0)),
    )
    return ln(w2.reshape(1, H), b2.reshape(1, H), summed)


def kernel(input_ids, position_ids, word_emb, pos_emb, tok_emb, ln_w, ln_b, fed_mask):
    ids = input_ids.reshape(-1).astype(jnp.int32)
    pids = position_ids.reshape(-1).astype(jnp.int32)
    pos2 = pos_emb + tok_emb[0]
    w2 = ln_w * fed_mask
    b2 = ln_b * fed_mask
    out = _run(ids, pids, word_emb, pos2, w2, b2)
    return out.reshape(B, S, H)
